# baseline (speedup 1.0000x reference)
import functools

import jax
import jax.numpy as jnp
from jax import lax
from jax.experimental import pallas as pl
from jax.experimental.pallas import tpu as pltpu
from jax.experimental.pallas import tpu_sc as plsc

_NEG_INF = float("-inf")

_NC = 2
_NS = 16
_L = 16
_NW = _NC * _NS


def _topk_body(nm, mb, g_ref, q_ref, k_ref, attn_ref, idx_ref, cv_ref, ci_ref):
    j = pl.program_id(0)
    s_rows = q_ref.shape[0]
    ncand = 4 * nm

    s = lax.dot_general(
        q_ref[...], k_ref[...], (((1,), (1,)), ((), ())),
        preferred_element_type=jnp.float32,
    )

    iota_f = lax.broadcasted_iota(jnp.int32, (s_rows, mb), 1).astype(jnp.float32)
    mbf = jnp.float32(mb)
    jbase = (j * mb).astype(jnp.float32)

    iota_c = lax.broadcasted_iota(jnp.int32, (s_rows, ncand), 1)
    cv = cv_ref[...]
    ci = ci_ref[...]
    for r in range(4):
        m = jnp.max(s, axis=1, keepdims=True)
        eq = s == m
        pf = jnp.min(jnp.where(eq, iota_f, mbf), axis=1, keepdims=True)
        if r < 3:
            s = jnp.where(iota_f == pf, _NEG_INF, s)
        col = iota_c == 4 * j + r
        cv = jnp.where(col, jnp.broadcast_to(m, cv.shape), cv)
        ci = jnp.where(col, jnp.broadcast_to(pf + jbase, ci.shape), ci)
    cv_ref[...] = cv
    ci_ref[...] = ci

    @pl.when(j == nm - 1)
    def _():
        cv = cv_ref[...]
        ci = ci_ref[...]
        iota_cf = lax.broadcasted_iota(
            jnp.int32, (s_rows, ncand), 1).astype(jnp.float32)
        iota4 = lax.broadcasted_iota(jnp.int32, (s_rows, 4), 1)
        tv = jnp.zeros((s_rows, 4), jnp.float32)
        ti = jnp.zeros((s_rows, 4), jnp.float32)
        for r in range(4):
            m = jnp.max(cv, axis=1, keepdims=True)
            pf = jnp.min(jnp.where(cv == m, iota_cf, float(ncand)),
                         axis=1, keepdims=True)
            sel = iota_cf == pf
            iv = jnp.sum(jnp.where(sel, ci, 0.0), axis=1, keepdims=True)
            if r < 3:
                cv = jnp.where(sel, _NEG_INF, cv)
            tv = jnp.where(iota4 == r, jnp.broadcast_to(m, tv.shape), tv)
            ti = jnp.where(iota4 == r, jnp.broadcast_to(iv, ti.shape), ti)
        d = q_ref.shape[1]
        scale = 1.0 / (jnp.float32(d) ** 0.5)
        tvs = tv * scale
        mx = jnp.max(tvs, axis=1, keepdims=True)
        e = jnp.exp(tvs - mx)
        w = e / jnp.sum(e, axis=1, keepdims=True)
        attn_ref[...] = w * g_ref[...]
        idx_ref[...] = ti.astype(jnp.int32)


def _topk_attn(qs, mk, g):
    s_rows, d = qs.shape
    m_rows = mk.shape[0]
    mb = min(2048, m_rows)
    nm = m_rows // mb
    return pl.pallas_call(
        functools.partial(_topk_body, nm, mb),
        grid=(nm,),
        in_specs=[
            pl.BlockSpec((1, 1), lambda j: (0, 0)),
            pl.BlockSpec((s_rows, d), lambda j: (0, 0)),
            pl.BlockSpec((mb, d), lambda j: (j, 0)),
        ],
        out_specs=[
            pl.BlockSpec((s_rows, 4), lambda j: (0, 0)),
            pl.BlockSpec((s_rows, 4), lambda j: (0, 0)),
        ],
        out_shape=[
            jax.ShapeDtypeStruct((s_rows, 4), jnp.float32),
            jax.ShapeDtypeStruct((s_rows, 4), jnp.int32),
        ],
        scratch_shapes=[
            pltpu.VMEM((s_rows, 4 * nm), jnp.float32),
            pltpu.VMEM((s_rows, 4 * nm), jnp.float32),
        ],
        compiler_params=pltpu.CompilerParams(
            dimension_semantics=("arbitrary",),
        ),
    )(g.reshape(1, 1), qs, mk)


def _sc_combine(mv, idx_flat, attn_exp, lo):
    s_rows, d = lo.shape
    rows_per_w = s_rows // _NW
    ch = 8
    n_chunks = rows_per_w // ch
    mesh = plsc.VectorSubcoreMesh(core_axis_name="c", subcore_axis_name="s")

    @functools.partial(
        pl.kernel,
        mesh=mesh,
        out_type=jax.ShapeDtypeStruct((s_rows, d), jnp.float32),
        scratch_types=[
            pltpu.VMEM((2, ch * 4), jnp.int32),
            pltpu.VMEM((2, ch * 4, d), jnp.float32),
            pltpu.VMEM((ch * 4, _L), jnp.float32),
            pltpu.VMEM((ch, d), jnp.float32),
            pltpu.VMEM((ch, d), jnp.float32),
            pltpu.SemaphoreType.DMA,
            pltpu.SemaphoreType.DMA,
        ],
    )
    def k(mv_hbm, idx_hbm, attn_hbm, lo_hbm, out_hbm,
          idx_v, rows_v, attn_v, lo_v, out_v, sem0, sem1):
        wid = lax.axis_index("s") * _NC + lax.axis_index("c")
        base = wid * rows_per_w
        sems = (sem0, sem1)

        def issue(c):
            slot = c % 2
            ib = (base + c * ch) * 4
            pltpu.sync_copy(idx_hbm.at[pl.ds(ib, ch * 4)], idx_v.at[slot])
            return pltpu.async_copy(
                mv_hbm.at[idx_v.at[slot]], rows_v.at[slot], sems[slot])

        handles = [issue(0)]
        for c in range(n_chunks):
            slot = c % 2
            if c + 1 < n_chunks:
                handles.append(issue(c + 1))
            qbase = base + c * ch
            ibase = qbase * 4
            pltpu.sync_copy(attn_hbm.at[pl.ds(ibase, ch * 4)], attn_v)
            pltpu.sync_copy(lo_hbm.at[pl.ds(qbase, ch)], lo_v)
            handles[c].wait()
            rows_c = rows_v.at[slot]

            @pl.loop(0, ch)
            def _(w):
                wv0 = attn_v.at[pl.ds(4 * w + 0, 1), :][...]
                wv1 = attn_v.at[pl.ds(4 * w + 1, 1), :][...]
                wv2 = attn_v.at[pl.ds(4 * w + 2, 1), :][...]
                wv3 = attn_v.at[pl.ds(4 * w + 3, 1), :][...]

                @pl.loop(0, d, step=_L)
                def _(col):
                    sl = pl.ds(col, _L)
                    acc = lo_v.at[pl.ds(w, 1), sl][...]
                    acc = acc + wv0 * rows_c.at[pl.ds(4 * w + 0, 1), sl][...]
                    acc = acc + wv1 * rows_c.at[pl.ds(4 * w + 1, 1), sl][...]
                    acc = acc + wv2 * rows_c.at[pl.ds(4 * w + 2, 1), sl][...]
                    acc = acc + wv3 * rows_c.at[pl.ds(4 * w + 3, 1), sl][...]
                    out_v.at[pl.ds(w, 1), sl][...] = acc

            pltpu.sync_copy(out_v, out_hbm.at[pl.ds(qbase, ch)])

    return k(mv, idx_flat, attn_exp, lo)


def kernel(q, local_out, mem_k, mem_v, g):
    b, s_rows, d = q.shape
    qs = q.reshape(s_rows, d)
    mk = mem_k.reshape(-1, d)
    mv = mem_v.reshape(-1, d)
    lo = local_out.reshape(s_rows, d)

    nh = 2
    sh = s_rows // nh
    outs = []
    for h in range(nh):
        qh = lax.slice_in_dim(qs, h * sh, (h + 1) * sh, axis=0)
        loh = lax.slice_in_dim(lo, h * sh, (h + 1) * sh, axis=0)
        attn, idx = _topk_attn(qh, mk, g)
        attn_exp = jnp.broadcast_to(attn.reshape(sh * 4, 1), (sh * 4, _L))
        idx_flat = idx.reshape(sh * 4)
        outs.append(_sc_combine(mv, idx_flat, attn_exp, loh))
    out = jnp.concatenate(outs, axis=0)
    return out.reshape(b, s_rows, d)


# speedup vs baseline: 1.0413x; 1.0413x over previous
import functools

import jax
import jax.numpy as jnp
from jax import lax
from jax.experimental import pallas as pl
from jax.experimental.pallas import tpu as pltpu
from jax.experimental.pallas import tpu_sc as plsc

_NEG_INF = float("-inf")

_NC = 2
_NS = 16
_L = 16
_NW = _NC * _NS


def _topk_body(nm, mb, g_ref, q_ref, k0_ref, k1_ref,
               attn_ref, idx_ref, cv_ref, ci_ref):
    j = pl.program_id(0)
    s_rows = q_ref.shape[0]
    ncand = 8 * nm

    s0 = lax.dot_general(
        q_ref[...], k0_ref[...], (((1,), (1,)), ((), ())),
        preferred_element_type=jnp.float32,
    )
    s1 = lax.dot_general(
        q_ref[...], k1_ref[...], (((1,), (1,)), ((), ())),
        preferred_element_type=jnp.float32,
    )

    iota_f = lax.broadcasted_iota(jnp.int32, (s_rows, mb), 1).astype(jnp.float32)
    mbf = jnp.float32(mb)
    iota_c = lax.broadcasted_iota(jnp.int32, (s_rows, ncand), 1)

    cv = cv_ref[...]
    ci = ci_ref[...]
    for h, s in ((0, s0), (1, s1)):
        basef = ((2 * j + h) * mb).astype(jnp.float32)
        for r in range(4):
            m = jnp.max(s, axis=1, keepdims=True)
            eq = s == m
            pf = jnp.min(jnp.where(eq, iota_f, mbf), axis=1, keepdims=True)
            if r < 3:
                s = jnp.where(iota_f == pf, _NEG_INF, s)
            col = iota_c == 8 * j + 4 * h + r
            cv = jnp.where(col, jnp.broadcast_to(m, cv.shape), cv)
            ci = jnp.where(col, jnp.broadcast_to(pf + basef, ci.shape), ci)
    cv_ref[...] = cv
    ci_ref[...] = ci

    @pl.when(j == nm - 1)
    def _():
        cv = cv_ref[...]
        ci = ci_ref[...]
        iota_cf = lax.broadcasted_iota(
            jnp.int32, (s_rows, ncand), 1).astype(jnp.float32)
        iota4 = lax.broadcasted_iota(jnp.int32, (s_rows, 4), 1)
        tv = jnp.zeros((s_rows, 4), jnp.float32)
        ti = jnp.zeros((s_rows, 4), jnp.float32)
        for r in range(4):
            m = jnp.max(cv, axis=1, keepdims=True)
            pf = jnp.min(jnp.where(cv == m, iota_cf, float(ncand)),
                         axis=1, keepdims=True)
            sel = iota_cf == pf
            iv = jnp.sum(jnp.where(sel, ci, 0.0), axis=1, keepdims=True)
            if r < 3:
                cv = jnp.where(sel, _NEG_INF, cv)
            tv = jnp.where(iota4 == r, jnp.broadcast_to(m, tv.shape), tv)
            ti = jnp.where(iota4 == r, jnp.broadcast_to(iv, ti.shape), ti)
        d = q_ref.shape[1]
        scale = 1.0 / (jnp.float32(d) ** 0.5)
        tvs = tv * scale
        mx = jnp.max(tvs, axis=1, keepdims=True)
        e = jnp.exp(tvs - mx)
        w = e / jnp.sum(e, axis=1, keepdims=True)
        attn_ref[...] = w * g_ref[...]
        idx_ref[...] = ti.astype(jnp.int32)


def _topk_attn(qs, mk, g):
    s_rows, d = qs.shape
    m_rows = mk.shape[0]
    mb = min(1024, m_rows)
    nm = max(m_rows // (2 * mb), 1)
    return pl.pallas_call(
        functools.partial(_topk_body, nm, mb),
        grid=(nm,),
        in_specs=[
            pl.BlockSpec((1, 1), lambda j: (0, 0)),
            pl.BlockSpec((s_rows, d), lambda j: (0, 0)),
            pl.BlockSpec((mb, d), lambda j: (2 * j, 0)),
            pl.BlockSpec((mb, d), lambda j: (2 * j + 1, 0)),
        ],
        out_specs=[
            pl.BlockSpec((s_rows, 4), lambda j: (0, 0)),
            pl.BlockSpec((s_rows, 4), lambda j: (0, 0)),
        ],
        out_shape=[
            jax.ShapeDtypeStruct((s_rows, 4), jnp.float32),
            jax.ShapeDtypeStruct((s_rows, 4), jnp.int32),
        ],
        scratch_shapes=[
            pltpu.VMEM((s_rows, 8 * nm), jnp.float32),
            pltpu.VMEM((s_rows, 8 * nm), jnp.float32),
        ],
        compiler_params=pltpu.CompilerParams(
            dimension_semantics=("arbitrary",),
        ),
    )(g.reshape(1, 1), qs, mk, mk)


def _sc_combine(mv, idx_flat, attn_exp, lo):
    s_rows, d = lo.shape
    rows_per_w = s_rows // _NW
    ch = 8
    n_chunks = rows_per_w // ch
    mesh = plsc.VectorSubcoreMesh(core_axis_name="c", subcore_axis_name="s")

    @functools.partial(
        pl.kernel,
        mesh=mesh,
        out_type=jax.ShapeDtypeStruct((s_rows, d), jnp.float32),
        scratch_types=[
            pltpu.VMEM((2, ch * 4), jnp.int32),
            pltpu.VMEM((2, ch * 4, d), jnp.float32),
            pltpu.VMEM((ch * 4, _L), jnp.float32),
            pltpu.VMEM((ch, d), jnp.float32),
            pltpu.VMEM((ch, d), jnp.float32),
            pltpu.SemaphoreType.DMA,
            pltpu.SemaphoreType.DMA,
        ],
    )
    def k(mv_hbm, idx_hbm, attn_hbm, lo_hbm, out_hbm,
          idx_v, rows_v, attn_v, lo_v, out_v, sem0, sem1):
        wid = lax.axis_index("s") * _NC + lax.axis_index("c")
        base = wid * rows_per_w
        sems = (sem0, sem1)

        def issue(c):
            slot = c % 2
            ib = (base + c * ch) * 4
            pltpu.sync_copy(idx_hbm.at[pl.ds(ib, ch * 4)], idx_v.at[slot])
            return pltpu.async_copy(
                mv_hbm.at[idx_v.at[slot]], rows_v.at[slot], sems[slot])

        handles = [issue(0)]
        for c in range(n_chunks):
            slot = c % 2
            if c + 1 < n_chunks:
                handles.append(issue(c + 1))
            qbase = base + c * ch
            ibase = qbase * 4
            pltpu.sync_copy(attn_hbm.at[pl.ds(ibase, ch * 4)], attn_v)
            pltpu.sync_copy(lo_hbm.at[pl.ds(qbase, ch)], lo_v)
            handles[c].wait()
            rows_c = rows_v.at[slot]

            @pl.loop(0, ch)
            def _(w):
                wv0 = attn_v.at[pl.ds(4 * w + 0, 1), :][...]
                wv1 = attn_v.at[pl.ds(4 * w + 1, 1), :][...]
                wv2 = attn_v.at[pl.ds(4 * w + 2, 1), :][...]
                wv3 = attn_v.at[pl.ds(4 * w + 3, 1), :][...]

                @pl.loop(0, d, step=_L)
                def _(col):
                    sl = pl.ds(col, _L)
                    acc = lo_v.at[pl.ds(w, 1), sl][...]
                    acc = acc + wv0 * rows_c.at[pl.ds(4 * w + 0, 1), sl][...]
                    acc = acc + wv1 * rows_c.at[pl.ds(4 * w + 1, 1), sl][...]
                    acc = acc + wv2 * rows_c.at[pl.ds(4 * w + 2, 1), sl][...]
                    acc = acc + wv3 * rows_c.at[pl.ds(4 * w + 3, 1), sl][...]
                    out_v.at[pl.ds(w, 1), sl][...] = acc

            pltpu.sync_copy(out_v, out_hbm.at[pl.ds(qbase, ch)])

    return k(mv, idx_flat, attn_exp, lo)


def kernel(q, local_out, mem_k, mem_v, g):
    b, s_rows, d = q.shape
    qs = q.reshape(s_rows, d)
    mk = mem_k.reshape(-1, d)
    mv = mem_v.reshape(-1, d)
    lo = local_out.reshape(s_rows, d)

    attn, idx = _topk_attn(qs, mk, g)
    attn_exp = jnp.broadcast_to(attn.reshape(s_rows * 4, 1), (s_rows * 4, _L))
    idx_flat = idx.reshape(s_rows * 4)
    out = _sc_combine(mv, idx_flat, attn_exp, lo)
    return out.reshape(b, s_rows, d)


# speedup vs baseline: 1.0424x; 1.0010x over previous
import functools

import jax
import jax.numpy as jnp
from jax import lax
from jax.experimental import pallas as pl
from jax.experimental.pallas import tpu as pltpu
from jax.experimental.pallas import tpu_sc as plsc

_NEG_INF = float("-inf")

_NC = 2
_NS = 16
_L = 16
_NW = _NC * _NS


def _topk_body(nm, mb, nsb, g_ref, q_ref, *refs):
    k_refs = refs[:nsb]
    attn_ref, idx_ref, cv_ref, ci_ref = refs[nsb:]
    j = pl.program_id(0)
    s_rows = q_ref.shape[0]
    ncand = 4 * nsb * nm

    subs = [
        lax.dot_general(
            q_ref[...], kr[...], (((1,), (1,)), ((), ())),
            preferred_element_type=jnp.float32,
        )
        for kr in k_refs
    ]

    iota_f = lax.broadcasted_iota(jnp.int32, (s_rows, mb), 1).astype(jnp.float32)
    mbf = jnp.float32(mb)
    iota_c = lax.broadcasted_iota(jnp.int32, (s_rows, ncand), 1)

    cv = cv_ref[...]
    ci = ci_ref[...]
    for h, s in enumerate(subs):
        basef = ((nsb * j + h) * mb).astype(jnp.float32)
        for r in range(4):
            m = jnp.max(s, axis=1, keepdims=True)
            eq = s == m
            pf = jnp.min(jnp.where(eq, iota_f, mbf), axis=1, keepdims=True)
            if r < 3:
                s = jnp.where(iota_f == pf, _NEG_INF, s)
            col = iota_c == 4 * (nsb * j + h) + r
            cv = jnp.where(col, jnp.broadcast_to(m, cv.shape), cv)
            ci = jnp.where(col, jnp.broadcast_to(pf + basef, ci.shape), ci)
    cv_ref[...] = cv
    ci_ref[...] = ci

    @pl.when(j == nm - 1)
    def _():
        cv = cv_ref[...]
        ci = ci_ref[...]
        iota_cf = lax.broadcasted_iota(
            jnp.int32, (s_rows, ncand), 1).astype(jnp.float32)
        iota4 = lax.broadcasted_iota(jnp.int32, (s_rows, 4), 1)
        tv = jnp.zeros((s_rows, 4), jnp.float32)
        ti = jnp.zeros((s_rows, 4), jnp.float32)
        for r in range(4):
            m = jnp.max(cv, axis=1, keepdims=True)
            pf = jnp.min(jnp.where(cv == m, iota_cf, float(ncand)),
                         axis=1, keepdims=True)
            sel = iota_cf == pf
            iv = jnp.sum(jnp.where(sel, ci, 0.0), axis=1, keepdims=True)
            if r < 3:
                cv = jnp.where(sel, _NEG_INF, cv)
            tv = jnp.where(iota4 == r, jnp.broadcast_to(m, tv.shape), tv)
            ti = jnp.where(iota4 == r, jnp.broadcast_to(iv, ti.shape), ti)
        d = q_ref.shape[1]
        scale = 1.0 / (jnp.float32(d) ** 0.5)
        tvs = tv * scale
        mx = jnp.max(tvs, axis=1, keepdims=True)
        e = jnp.exp(tvs - mx)
        w = e / jnp.sum(e, axis=1, keepdims=True)
        attn_ref[...] = w * g_ref[...]
        idx_ref[...] = ti.astype(jnp.int32)


def _topk_attn(qs, mk, g):
    s_rows, d = qs.shape
    m_rows = mk.shape[0]
    nsb = 4
    mb = min(512, m_rows)
    nm = max(m_rows // (nsb * mb), 1)

    def kspec(h):
        return pl.BlockSpec((mb, d), lambda j, h=h: (nsb * j + h, 0))

    return pl.pallas_call(
        functools.partial(_topk_body, nm, mb, nsb),
        grid=(nm,),
        in_specs=[
            pl.BlockSpec((1, 1), lambda j: (0, 0)),
            pl.BlockSpec((s_rows, d), lambda j: (0, 0)),
        ] + [kspec(h) for h in range(nsb)],
        out_specs=[
            pl.BlockSpec((s_rows, 4), lambda j: (0, 0)),
            pl.BlockSpec((s_rows, 4), lambda j: (0, 0)),
        ],
        out_shape=[
            jax.ShapeDtypeStruct((s_rows, 4), jnp.float32),
            jax.ShapeDtypeStruct((s_rows, 4), jnp.int32),
        ],
        scratch_shapes=[
            pltpu.VMEM((s_rows, 4 * nsb * nm), jnp.float32),
            pltpu.VMEM((s_rows, 4 * nsb * nm), jnp.float32),
        ],
        compiler_params=pltpu.CompilerParams(
            dimension_semantics=("arbitrary",),
        ),
    )(g.reshape(1, 1), qs, *([mk] * nsb))


def _sc_combine(mv, idx_flat, attn_exp, lo):
    s_rows, d = lo.shape
    rows_per_w = s_rows // _NW
    ch = 8
    n_chunks = rows_per_w // ch
    mesh = plsc.VectorSubcoreMesh(core_axis_name="c", subcore_axis_name="s")

    @functools.partial(
        pl.kernel,
        mesh=mesh,
        out_type=jax.ShapeDtypeStruct((s_rows, d), jnp.float32),
        scratch_types=[
            pltpu.VMEM((2, ch * 4), jnp.int32),
            pltpu.VMEM((2, ch * 4, d), jnp.float32),
            pltpu.VMEM((ch * 4, _L), jnp.float32),
            pltpu.VMEM((ch, d), jnp.float32),
            pltpu.VMEM((ch, d), jnp.float32),
            pltpu.SemaphoreType.DMA,
            pltpu.SemaphoreType.DMA,
        ],
    )
    def k(mv_hbm, idx_hbm, attn_hbm, lo_hbm, out_hbm,
          idx_v, rows_v, attn_v, lo_v, out_v, sem0, sem1):
        wid = lax.axis_index("s") * _NC + lax.axis_index("c")
        base = wid * rows_per_w
        sems = (sem0, sem1)

        def issue(c):
            slot = c % 2
            ib = (base + c * ch) * 4
            pltpu.sync_copy(idx_hbm.at[pl.ds(ib, ch * 4)], idx_v.at[slot])
            return pltpu.async_copy(
                mv_hbm.at[idx_v.at[slot]], rows_v.at[slot], sems[slot])

        handles = [issue(0)]
        for c in range(n_chunks):
            slot = c % 2
            if c + 1 < n_chunks:
                handles.append(issue(c + 1))
            qbase = base + c * ch
            ibase = qbase * 4
            pltpu.sync_copy(attn_hbm.at[pl.ds(ibase, ch * 4)], attn_v)
            pltpu.sync_copy(lo_hbm.at[pl.ds(qbase, ch)], lo_v)
            handles[c].wait()
            rows_c = rows_v.at[slot]

            @pl.loop(0, ch)
            def _(w):
                wv0 = attn_v.at[pl.ds(4 * w + 0, 1), :][...]
                wv1 = attn_v.at[pl.ds(4 * w + 1, 1), :][...]
                wv2 = attn_v.at[pl.ds(4 * w + 2, 1), :][...]
                wv3 = attn_v.at[pl.ds(4 * w + 3, 1), :][...]

                @pl.loop(0, d, step=_L)
                def _(col):
                    sl = pl.ds(col, _L)
                    acc = lo_v.at[pl.ds(w, 1), sl][...]
                    acc = acc + wv0 * rows_c.at[pl.ds(4 * w + 0, 1), sl][...]
                    acc = acc + wv1 * rows_c.at[pl.ds(4 * w + 1, 1), sl][...]
                    acc = acc + wv2 * rows_c.at[pl.ds(4 * w + 2, 1), sl][...]
                    acc = acc + wv3 * rows_c.at[pl.ds(4 * w + 3, 1), sl][...]
                    out_v.at[pl.ds(w, 1), sl][...] = acc

            pltpu.sync_copy(out_v, out_hbm.at[pl.ds(qbase, ch)])

    return k(mv, idx_flat, attn_exp, lo)


def kernel(q, local_out, mem_k, mem_v, g):
    b, s_rows, d = q.shape
    qs = q.reshape(s_rows, d)
    mk = mem_k.reshape(-1, d)
    mv = mem_v.reshape(-1, d)
    lo = local_out.reshape(s_rows, d)

    attn, idx = _topk_attn(qs, mk, g)
    attn_exp = jnp.broadcast_to(attn.reshape(s_rows * 4, 1), (s_rows * 4, _L))
    idx_flat = idx.reshape(s_rows * 4)
    out = _sc_combine(mv, idx_flat, attn_exp, lo)
    return out.reshape(b, s_rows, d)
